# baseline (speedup 1.0000x reference)
import jax
import jax.numpy as jnp
from jax.experimental import pallas as pl

_SEQ_BLK = 256


def _bcast_body(w_ref, o_ref):
    o_ref[...] = jnp.broadcast_to(w_ref[...][None, :, :], o_ref.shape)


def kernel(tokens, W_pos):
    batch, seq = tokens.shape
    d = W_pos.shape[-1]
    blk = min(_SEQ_BLK, seq)
    grid = (seq // blk,)
    return pl.pallas_call(
        _bcast_body,
        grid=grid,
        in_specs=[pl.BlockSpec((blk, d), lambda i: (i, 0))],
        out_specs=pl.BlockSpec((batch, blk, d), lambda i: (0, i, 0)),
        out_shape=jax.ShapeDtypeStruct((batch, seq, d), W_pos.dtype),
    )(W_pos[:seq])


# speedup vs baseline: 1.2490x; 1.2490x over previous
import jax
import jax.numpy as jnp
from jax.experimental import pallas as pl
from jax.experimental.pallas import tpu as pltpu

_N_CHUNKS = 4


def _make_body(batch, seq, d):
    rows = seq // _N_CHUNKS

    def body(w_hbm, out_hbm, w_vmem, in_sems, out_sems):
        in_cps = []
        for c in range(_N_CHUNKS):
            sl = pl.ds(c * rows, rows)
            cp = pltpu.make_async_copy(
                w_hbm.at[sl, :], w_vmem.at[sl, :], in_sems.at[c])
            cp.start()
            in_cps.append(cp)
        out_cps = []
        for c in range(_N_CHUNKS):
            in_cps[c].wait()
            sl = pl.ds(c * rows, rows)
            for b in range(batch):
                cp = pltpu.make_async_copy(
                    w_vmem.at[sl, :], out_hbm.at[b, sl, :], out_sems.at[b, c])
                cp.start()
                out_cps.append(cp)
        for cp in out_cps:
            cp.wait()

    return body


def kernel(tokens, W_pos):
    batch, seq = tokens.shape
    d = W_pos.shape[-1]
    return pl.pallas_call(
        _make_body(batch, seq, d),
        in_specs=[pl.BlockSpec(memory_space=pltpu.MemorySpace.HBM)],
        out_specs=pl.BlockSpec(memory_space=pltpu.MemorySpace.HBM),
        out_shape=jax.ShapeDtypeStruct((batch, seq, d), W_pos.dtype),
        scratch_shapes=[
            pltpu.VMEM((seq, d), W_pos.dtype),
            pltpu.SemaphoreType.DMA((_N_CHUNKS,)),
            pltpu.SemaphoreType.DMA((batch, _N_CHUNKS)),
        ],
    )(W_pos[:seq])
